# baseline (speedup 1.0000x reference)
import functools

import jax
import jax.numpy as jnp
from jax import lax
from jax.experimental import pallas as pl
from jax.experimental.pallas import tpu as pltpu
from jax.experimental.pallas import tpu_sc as plsc

VOCAB = 1000000
SEQ = 200
DIM = 64
BATCH = 1024
EPS = 1e-05

NC = 2
NS = 16
NW = NC * NS
L = 16

TOTAL = BATCH * SEQ
RPW = TOTAL // NW
CHUNK = 128
NCHUNK = RPW // CHUNK


def _rsqrt(x):
    i = lax.bitcast_convert_type(x, jnp.int32)
    i = jnp.full((L,), 0x5F3759DF, jnp.int32) - lax.shift_right_arithmetic(
        i, jnp.full((L,), 1, jnp.int32))
    y = lax.bitcast_convert_type(i, jnp.float32)
    y = y * (1.5 - 0.5 * x * y * y)
    y = y * (1.5 - 0.5 * x * y * y)
    return y


_GATHER_DNUMS = lax.GatherDimensionNumbers(
    offset_dims=(), collapsed_slice_dims=(0,), start_index_map=(0,))


def _shuffle(x, idx):
    return lax.gather(x, idx[:, None], _GATHER_DNUMS, (1,),
                      mode=lax.GatherScatterMode.PROMISE_IN_BOUNDS)


def _hsum(x):
    for sh in (8, 4, 2, 1):
        idx = lax.iota(jnp.int32, L) ^ sh
        x = x + _shuffle(x, idx)
    return x


def _body(state_hbm, token_hbm, pos_hbm, gamma_hbm, beta_hbm, out_hbm,
          idx_v, rows_v, pos_v, gamma_v, beta_v, sem):
    wid = lax.axis_index("s") * NC + lax.axis_index("c")

    pltpu.sync_copy(state_hbm.at[wid], idx_v)
    pltpu.sync_copy(pos_hbm, pos_v)
    pltpu.sync_copy(gamma_hbm, gamma_v)
    pltpu.sync_copy(beta_hbm, beta_v)

    g = [gamma_v[pl.ds(k * L, L)] for k in range(4)]
    b = [beta_v[pl.ds(k * L, L)] for k in range(4)]

    def chunk_step(c, carry):
        base = wid * RPW + c * CHUNK
        start = lax.rem(c * CHUNK, SEQ)

        pltpu.async_copy(token_hbm.at[idx_v.at[c]], rows_v, sem).wait()

        def row_step(r, carry2):
            pp = start + r
            pp = jnp.where(pp >= SEQ, pp - SEQ, pp)
            x = [rows_v[r, pl.ds(k * L, L)] + pos_v[pp, pl.ds(k * L, L)]
                 for k in range(4)]
            tot = _hsum((x[0] + x[1]) + (x[2] + x[3]))
            qtot = _hsum((x[0] * x[0] + x[1] * x[1])
                         + (x[2] * x[2] + x[3] * x[3]))
            mean = tot * (1.0 / DIM)
            var = qtot * (1.0 / DIM) - mean * mean
            rstd = _rsqrt(var + EPS)
            for k in range(4):
                rows_v[r, pl.ds(k * L, L)] = (x[k] - mean) * rstd * g[k] + b[k]
            return carry2

        lax.fori_loop(0, CHUNK, row_step, 0, unroll=2)

        pltpu.sync_copy(rows_v, out_hbm.at[pl.ds(base, CHUNK)])
        return carry

    lax.fori_loop(0, NCHUNK, chunk_step, 0)


@jax.jit
def _run(state3d, token_table, pos_table, gamma, beta):
    mesh = plsc.VectorSubcoreMesh(core_axis_name="c", subcore_axis_name="s",
                                  num_cores=NC, num_subcores=NS)
    f = pl.kernel(
        _body,
        out_type=jax.ShapeDtypeStruct((TOTAL, DIM), jnp.float32),
        mesh=mesh,
        scratch_types=[
            pltpu.VMEM((NCHUNK, CHUNK), jnp.int32),
            pltpu.VMEM((CHUNK, DIM), jnp.float32),
            pltpu.VMEM((SEQ, DIM), jnp.float32),
            pltpu.VMEM((DIM,), jnp.float32),
            pltpu.VMEM((DIM,), jnp.float32),
            pltpu.SemaphoreType.DMA,
        ],
        compiler_params=pltpu.CompilerParams(use_tc_tiling_on_sc=False),
    )
    return f(state3d, token_table, pos_table, gamma, beta)


def kernel(state, token_table, pos_table, gamma, beta):
    state3d = state.reshape(NW, NCHUNK, CHUNK).astype(jnp.int32)
    out = _run(state3d, token_table, pos_table, gamma, beta)
    return out.reshape(BATCH, SEQ, DIM)


# speedup vs baseline: 1.1837x; 1.1837x over previous
import jax
import jax.numpy as jnp
from jax import lax
from jax.experimental import pallas as pl
from jax.experimental.pallas import tpu as pltpu
from jax.experimental.pallas import tpu_sc as plsc

VOCAB = 1000000
SEQ = 200
DIM = 64
BATCH = 1024
EPS = 1e-05

NC = 2
NS = 16
NW = NC * NS
L = 16

SPW = BATCH // NW
G = 16
NG = (SEQ + G - 1) // G


def _group_rows(gi):
    return G if (gi + 1) * G <= SEQ else SEQ - gi * G

_GATHER_DNUMS = lax.GatherDimensionNumbers(
    offset_dims=(), collapsed_slice_dims=(0,), start_index_map=(0,))


def _shuffle(x, idx):
    return lax.gather(x, idx[:, None], _GATHER_DNUMS, (1,),
                      mode=lax.GatherScatterMode.PROMISE_IN_BOUNDS)


def _hsum(x):
    for sh in (8, 4, 2, 1):
        idx = lax.iota(jnp.int32, L) ^ sh
        x = x + _shuffle(x, idx)
    return x


def _rsqrt(x):
    i = lax.bitcast_convert_type(x, jnp.int32)
    i = jnp.full((L,), 0x5F3759DF, jnp.int32) - lax.shift_right_arithmetic(
        i, jnp.full((L,), 1, jnp.int32))
    y = lax.bitcast_convert_type(i, jnp.float32)
    y = y * (1.5 - 0.5 * x * y * y)
    y = y * (1.5 - 0.5 * x * y * y)
    return y


def _body(state_hbm, token_hbm, pos_hbm, gb_hbm, out_hbm,
          idx_v, rows_v, out_v, pos_v, gb_v,
          gsem0, gsem1, osem):
    wid = lax.axis_index("s") * NC + lax.axis_index("c")

    pltpu.sync_copy(pos_hbm, pos_v)
    pltpu.sync_copy(gb_hbm, gb_v)

    g_vec = [gb_v[pl.ds(k * L, L)] for k in range(4)]
    b_vec = [gb_v[pl.ds(DIM + k * L, L)] for k in range(4)]
    gsems = [gsem0, gsem1]

    def fire_group(gi, buf):
        idx = idx_v[pl.ds(gi * G, G)]
        copies = []
        for j in range(_group_rows(gi)):
            copies.append(pltpu.async_copy(
                token_hbm.at[idx[j]], rows_v.at[buf, j], gsems[buf]))
        return copies

    def compute_group(gi, buf, copies):
        for c in copies:
            c.wait()
        for j in range(_group_rows(gi)):
            r = gi * G + j
            x = [rows_v[buf, j, pl.ds(k * L, L)]
                 + pos_v[pl.ds(r * DIM + k * L, L)] for k in range(4)]
            tot = _hsum((x[0] + x[1]) + (x[2] + x[3]))
            qtot = _hsum((x[0] * x[0] + x[1] * x[1])
                         + (x[2] * x[2] + x[3] * x[3]))
            mean = tot * (1.0 / DIM)
            var = qtot * (1.0 / DIM) - mean * mean
            rstd = _rsqrt(var + EPS)
            for k in range(4):
                out_v[r, pl.ds(k * L, L)] = ((x[k] - mean) * rstd * g_vec[k]
                                             + b_vec[k])

    def seq_step(c, carry):
        b = wid * SPW + c
        pltpu.sync_copy(state_hbm.at[pl.ds(b * SEQ, SEQ)],
                        idx_v.at[pl.ds(0, SEQ)])
        copies = fire_group(0, 0)
        for gi in range(NG):
            nxt = None
            if gi + 1 < NG:
                nxt = fire_group(gi + 1, (gi + 1) % 2)
            compute_group(gi, gi % 2, copies)
            copies = nxt
        pltpu.async_copy(out_v, out_hbm.at[b], osem).wait()
        return carry

    lax.fori_loop(0, SPW, seq_step, 0)


@jax.jit
def _run(state, token_table, pos_table, gb):
    mesh = plsc.VectorSubcoreMesh(core_axis_name="c", subcore_axis_name="s",
                                  num_cores=NC, num_subcores=NS)
    f = pl.kernel(
        _body,
        out_type=jax.ShapeDtypeStruct((BATCH, SEQ, DIM), jnp.float32),
        mesh=mesh,
        scratch_types=[
            pltpu.VMEM((NG * G,), jnp.int32),
            pltpu.VMEM((2, G, DIM), jnp.float32),
            pltpu.VMEM((SEQ, DIM), jnp.float32),
            pltpu.VMEM((SEQ * DIM,), jnp.float32),
            pltpu.VMEM((2 * DIM,), jnp.float32),
            pltpu.SemaphoreType.DMA,
            pltpu.SemaphoreType.DMA,
            pltpu.SemaphoreType.DMA,
        ],
        compiler_params=pltpu.CompilerParams(use_tc_tiling_on_sc=True),
    )
    return f(state, token_table, pos_table, gb)


def kernel(state, token_table, pos_table, gamma, beta):
    gb = jnp.concatenate([gamma, beta])
    return _run(state.reshape(-1).astype(jnp.int32), token_table,
                pos_table.reshape(-1), gb)
